# baseline (speedup 1.0000x reference)
import functools

import jax
import jax.numpy as jnp
from jax import lax
from jax.experimental import pallas as pl
from jax.experimental.pallas import tpu as pltpu
from jax.experimental.pallas import tpu_sc as plsc

N_PATCHES = 1024
MODEL_DIM = 768
BATCH = 64

NC = 2
NS = 16
NW = NC * NS
RPW = N_PATCHES // NW
HB = BATCH // 2
NJ = MODEL_DIM // 16

_mesh = plsc.VectorSubcoreMesh(core_axis_name="c", subcore_axis_name="s")


@functools.partial(
    pl.kernel,
    mesh=_mesh,
    out_type=jax.ShapeDtypeStruct((BATCH, N_PATCHES, MODEL_DIM), jnp.float32),
    scratch_types=[
        pltpu.VMEM((2, MODEL_DIM), jnp.float32),
        pltpu.VMEM((2, HB, MODEL_DIM), jnp.float32),
        pltpu.VMEM((2, HB, MODEL_DIM), jnp.float32),
        pltpu.SemaphoreType.DMA,
        pltpu.SemaphoreType.DMA,
        pltpu.SemaphoreType.DMA,
        pltpu.SemaphoreType.DMA,
        pltpu.SemaphoreType.DMA,
        pltpu.SemaphoreType.DMA,
    ],
)
def _sc_add(patches_hbm, pos_hbm, out_hbm, pos_v, in_v, out_v,
            si0, si1, so0, so1, sp0, sp1):
    wid = lax.axis_index("s") * NC + lax.axis_index("c")
    row0 = wid * RPW
    sin = (si0, si1)
    sout = (so0, so1)
    spos = (sp0, sp1)

    def in_cp(g, h):
        return pltpu.make_async_copy(
            patches_hbm.at[pl.ds(h * HB, HB), row0 + g, :], in_v.at[h], sin[h])

    def out_cp(g, h):
        return pltpu.make_async_copy(
            out_v.at[h], out_hbm.at[pl.ds(h * HB, HB), row0 + g, :], sout[h])

    def pos_cp(g, ps):
        return pltpu.make_async_copy(pos_hbm.at[row0 + g, :], pos_v.at[ps],
                                     spos[ps])

    def compute(h, ps):
        def body(j, c):
            for u in range(2):
                sl = pl.ds((j * 2 + u) * 16, 16)
                pvj = pos_v[ps, sl]
                for b in range(HB):
                    out_v[h, b, sl] = in_v[h, b, sl] + pvj
            return c
        lax.fori_loop(0, NJ // 2, body, 0)

    def do_row(g, ps, first):
        pos_cp(g, ps).wait()
        for h in (0, 1):
            in_cp(g, h).wait()
            if not first:
                out_cp(g - 1, h).wait()
            compute(h, ps)
            in_cp(jnp.minimum(g + 1, RPW - 1), h).start()
            out_cp(g, h).start()
        pos_cp(jnp.minimum(g + 2, RPW - 1), ps).start()

    in_cp(0, 0).start()
    in_cp(0, 1).start()
    pos_cp(0, 0).start()
    pos_cp(1, 1).start()

    do_row(0, 0, True)
    do_row(1, 1, False)

    def pair_step(G, carry):
        do_row(2 * G, 0, False)
        do_row(2 * G + 1, 1, False)
        return carry

    lax.fori_loop(1, RPW // 2, pair_step, 0)

    for h in (0, 1):
        out_cp(RPW - 1, h).wait()
        in_cp(RPW - 1, h).wait()
    pos_cp(RPW - 1, 0).wait()
    pos_cp(RPW - 1, 1).wait()


def kernel(patches, pos_table):
    return _sc_add(patches, pos_table)


# speedup vs baseline: 2.4643x; 2.4643x over previous
import functools

import jax
import jax.numpy as jnp
from jax import lax
from jax.experimental import pallas as pl
from jax.experimental.pallas import tpu as pltpu
from jax.experimental.pallas import tpu_sc as plsc

N_PATCHES = 1024
MODEL_DIM = 768
BATCH = 64

NC = 2
NS = 16
NW = NC * NS
RPW = N_PATCHES // NW
HB = BATCH // 2
NJ = MODEL_DIM // 16

_mesh = plsc.VectorSubcoreMesh(core_axis_name="c", subcore_axis_name="s")


@functools.partial(
    pl.kernel,
    mesh=_mesh,
    out_type=jax.ShapeDtypeStruct((BATCH, N_PATCHES, MODEL_DIM), jnp.float32),
    scratch_types=[
        pltpu.VMEM((2, MODEL_DIM), jnp.float32),
        pltpu.VMEM((2, HB, MODEL_DIM), jnp.float32),
        pltpu.VMEM((2, HB, MODEL_DIM), jnp.float32),
        pltpu.SemaphoreType.DMA,
        pltpu.SemaphoreType.DMA,
        pltpu.SemaphoreType.DMA,
        pltpu.SemaphoreType.DMA,
        pltpu.SemaphoreType.DMA,
        pltpu.SemaphoreType.DMA,
    ],
)
def _sc_add(patches_hbm, pos_hbm, out_hbm, pos_v, in_v, out_v,
            si0, si1, so0, so1, sp0, sp1):
    wid = lax.axis_index("s") * NC + lax.axis_index("c")
    row0 = wid * RPW
    sin = (si0, si1)
    sout = (so0, so1)
    spos = (sp0, sp1)

    def in_cp(g, h):
        return pltpu.make_async_copy(
            patches_hbm.at[pl.ds(h * HB, HB), row0 + g, :], in_v.at[h], sin[h])

    def out_cp(g, h):
        return pltpu.make_async_copy(
            out_v.at[h], out_hbm.at[pl.ds(h * HB, HB), row0 + g, :], sout[h])

    def pos_cp(g, ps):
        return pltpu.make_async_copy(pos_hbm.at[row0 + g, :], pos_v.at[ps],
                                     spos[ps])

    def compute(h, ps):
        def body(j, c):
            sl = pl.ds(j * 16, 16)
            pvj = pos_v[ps, sl]
            for b in range(HB):
                out_v[h, b, sl] = in_v[h, b, sl] + pvj
            return c
        lax.fori_loop(0, NJ, body, 0)

    def do_row(g, ps, first):
        pos_cp(g, ps).wait()
        for h in (0, 1):
            in_cp(g, h).wait()
            if not first:
                out_cp(g - 1, h).wait()
            compute(h, ps)
            in_cp(jnp.minimum(g + 1, RPW - 1), h).start()
            out_cp(g, h).start()
        pos_cp(jnp.minimum(g + 2, RPW - 1), ps).start()

    in_cp(0, 0).start()
    in_cp(0, 1).start()
    pos_cp(0, 0).start()
    pos_cp(1, 1).start()

    do_row(0, 0, True)
    do_row(1, 1, False)

    def pair_step(G, carry):
        do_row(2 * G, 0, False)
        do_row(2 * G + 1, 1, False)
        return carry

    lax.fori_loop(1, RPW // 2, pair_step, 0)

    for h in (0, 1):
        out_cp(RPW - 1, h).wait()
        in_cp(RPW - 1, h).wait()
    pos_cp(RPW - 1, 0).wait()
    pos_cp(RPW - 1, 1).wait()


def kernel(patches, pos_table):
    return _sc_add(patches, pos_table)
